# Initial kernel scaffold; baseline (speedup 1.0000x reference)
import jax
import jax.numpy as jnp
from jax import lax
from jax.experimental import pallas as pl
from jax.experimental.pallas import tpu as pltpu
from jax.experimental.pallas import tpu_sc as plsc

_VOCAB = 100000
_B = 4096
_M = 100
_TAB_PAD = 100016

_info = plsc.get_sparse_core_info()
_NC, _NS, _L = _info.num_cores, _info.num_subcores, _info.num_lanes
_NW = _NC * _NS
_ROWS = _B // _NW
_GROUPS = _ROWS // _L


def _sc_body(idx_hbm, tab_hbm, out_hbm, idx_v, tab_v, out_v, sem_t, sem_i):
    wid = lax.axis_index("s") * _NC + lax.axis_index("c")
    base = wid * _ROWS

    cp_tab = pltpu.async_copy(tab_hbm, tab_v, sem_t)
    cp_idx = pltpu.async_copy(
        idx_hbm.at[pl.ds(base * _M, _ROWS * _M)], idx_v, sem_i)
    cp_tab.wait()
    cp_idx.wait()

    lane_off = lax.iota(jnp.int32, _L) * _M
    for r in range(_GROUPS):
        pos0 = lane_off + (r * _L * _M)

        def inner(m, acc):
            ids = plsc.load_gather(idx_v, [pos0 + m])
            vals = plsc.load_gather(tab_v, [ids])
            return acc + jnp.where(ids < _VOCAB, vals, jnp.float32(0.0))

        acc = lax.fori_loop(0, _M, inner, jnp.zeros((_L,), jnp.float32))
        out_v[pl.ds(r * _L, _L)] = acc

    pltpu.sync_copy(out_v, out_hbm.at[pl.ds(base, _ROWS)])


@jax.jit
def _sc_call(idx_flat, tab_flat):
    mesh = plsc.VectorSubcoreMesh(core_axis_name="c", subcore_axis_name="s")
    return pl.kernel(
        _sc_body,
        mesh=mesh,
        out_type=jax.ShapeDtypeStruct((_B,), jnp.float32),
        scratch_types=[
            pltpu.VMEM((_ROWS * _M,), jnp.int32),
            pltpu.VMEM((_TAB_PAD,), jnp.float32),
            pltpu.VMEM((_ROWS,), jnp.float32),
            pltpu.SemaphoreType.DMA,
            pltpu.SemaphoreType.DMA,
        ],
    )(idx_flat, tab_flat)


def kernel(inputs, table):
    idx_flat = inputs.reshape(-1)
    tab_flat = jnp.pad(table.reshape(-1), (0, _TAB_PAD - (_VOCAB + 1)))
    out = _sc_call(idx_flat, tab_flat)
    return out[:, None]


# speedup vs baseline: 73.9732x; 73.9732x over previous
import jax
import jax.numpy as jnp
from jax import lax
from jax.experimental import pallas as pl
from jax.experimental.pallas import tpu as pltpu
from jax.experimental.pallas import tpu_sc as plsc

_VOCAB = 100000
_B = 4096
_M = 100
_TAB_PAD = 100016

_info = plsc.get_sparse_core_info()
_NC, _NS, _L = _info.num_cores, _info.num_subcores, _info.num_lanes
_NW = _NC * _NS
_ROWS = _B // _NW
_GROUPS = _ROWS // _L


def _sc_body(idx_hbm, tab_hbm, out_hbm, idx_v, tab_v, out_v, sem_t, sem_i):
    wid = lax.axis_index("s") * _NC + lax.axis_index("c")
    base = wid * _ROWS

    cp_tab = pltpu.async_copy(tab_hbm, tab_v, sem_t)
    cp_idx = pltpu.async_copy(
        idx_hbm.at[pl.ds(base * _M, _ROWS * _M)], idx_v, sem_i)
    cp_tab.wait()
    cp_idx.wait()

    lane_off = lax.iota(jnp.int32, _L) * _M
    for r in range(_GROUPS):
        pos0 = lane_off + (r * _L * _M)

        def inner(m, acc):
            ids = plsc.load_gather(idx_v, [pos0 + m])
            vals = plsc.load_gather(tab_v, [ids])
            return acc + jnp.where(ids < _VOCAB, vals, jnp.float32(0.0))

        acc = lax.fori_loop(0, _M, inner, jnp.zeros((_L,), jnp.float32))
        out_v[pl.ds(r * _L, _L)] = acc

    pltpu.sync_copy(out_v, out_hbm.at[pl.ds(base, _ROWS)])


@jax.jit
def _sc_call(idx_flat, tab_flat):
    mesh = plsc.VectorSubcoreMesh(core_axis_name="c", subcore_axis_name="s")
    return pl.kernel(
        _sc_body,
        mesh=mesh,
        out_type=jax.ShapeDtypeStruct((_B,), jnp.float32),
        compiler_params=pltpu.CompilerParams(needs_layout_passes=False),
        scratch_types=[
            pltpu.VMEM((_ROWS * _M,), jnp.int32),
            pltpu.VMEM((_TAB_PAD,), jnp.float32),
            pltpu.VMEM((_ROWS,), jnp.float32),
            pltpu.SemaphoreType.DMA,
            pltpu.SemaphoreType.DMA,
        ],
    )(idx_flat, tab_flat)


def kernel(inputs, table):
    idx_flat = inputs.reshape(-1)
    tab_flat = jnp.pad(table.reshape(-1), (0, _TAB_PAD - (_VOCAB + 1)))
    out = _sc_call(idx_flat, tab_flat)
    return out[:, None]


# speedup vs baseline: 80.9529x; 1.0944x over previous
import jax
import jax.numpy as jnp
from jax import lax
from jax.experimental import pallas as pl
from jax.experimental.pallas import tpu as pltpu
from jax.experimental.pallas import tpu_sc as plsc

_VOCAB = 100000
_B = 4096
_M = 100
_TAB_PAD = 100016

_info = plsc.get_sparse_core_info()
_NC, _NS, _L = _info.num_cores, _info.num_subcores, _info.num_lanes
_NW = _NC * _NS
_ROWS = _B // _NW
_GROUPS = _ROWS // _L


def _sc_body(idx_hbm, tab_hbm, out_hbm, idx_v, tab_v, out_v, sem_t, sem_i):
    wid = lax.axis_index("s") * _NC + lax.axis_index("c")
    base = wid * _ROWS

    cp_tab = pltpu.async_copy(tab_hbm, tab_v, sem_t)
    cp_idx = pltpu.async_copy(
        idx_hbm.at[pl.ds(base * _M, _ROWS * _M)], idx_v, sem_i)
    cp_tab.wait()
    cp_idx.wait()

    tab_v[pl.ds(_VOCAB, _L)] = jnp.zeros((_L,), jnp.float32)

    lane_off = lax.iota(jnp.int32, _L) * _M
    pos0 = [lane_off + (r * _L * _M) for r in range(_GROUPS)]

    def inner(m, accs):
        out = []
        for r in range(_GROUPS):
            ids = plsc.load_gather(idx_v, [pos0[r] + m])
            vals = plsc.load_gather(tab_v, [ids])
            out.append(accs[r] + vals)
        return tuple(out)

    accs = lax.fori_loop(
        0, _M, inner,
        tuple(jnp.zeros((_L,), jnp.float32) for _ in range(_GROUPS)))
    for r in range(_GROUPS):
        out_v[pl.ds(r * _L, _L)] = accs[r]

    pltpu.sync_copy(out_v, out_hbm.at[pl.ds(base, _ROWS)])


@jax.jit
def _sc_call(idx_flat, tab_flat):
    mesh = plsc.VectorSubcoreMesh(core_axis_name="c", subcore_axis_name="s")
    return pl.kernel(
        _sc_body,
        mesh=mesh,
        out_type=jax.ShapeDtypeStruct((_B,), jnp.float32),
        compiler_params=pltpu.CompilerParams(needs_layout_passes=False),
        scratch_types=[
            pltpu.VMEM((_ROWS * _M,), jnp.int32),
            pltpu.VMEM((_TAB_PAD,), jnp.float32),
            pltpu.VMEM((_ROWS,), jnp.float32),
            pltpu.SemaphoreType.DMA,
            pltpu.SemaphoreType.DMA,
        ],
    )(idx_flat, tab_flat)


def kernel(inputs, table):
    idx_flat = inputs.reshape(-1)
    tab_flat = jnp.pad(table.reshape(-1), (0, _TAB_PAD - (_VOCAB + 1)))
    out = _sc_call(idx_flat, tab_flat)
    return out[:, None]


# speedup vs baseline: 81.7220x; 1.0095x over previous
import jax
import jax.numpy as jnp
from jax import lax
from jax.experimental import pallas as pl
from jax.experimental.pallas import tpu as pltpu
from jax.experimental.pallas import tpu_sc as plsc

_VOCAB = 100000
_B = 4096
_M = 100

_info = plsc.get_sparse_core_info()
_NC, _NS, _L = _info.num_cores, _info.num_subcores, _info.num_lanes
_NW = _NC * _NS
_ROWS = _B // _NW
_GROUPS = _ROWS // _L


def _sc_body(idx_hbm, tab_hbm, out_hbm, idx_v, tab_v, out_v, sem_t, sem_i):
    wid = lax.axis_index("s") * _NC + lax.axis_index("c")
    base = wid * _ROWS

    cp_tab = pltpu.async_copy(tab_hbm, tab_v, sem_t)
    cp_idx = pltpu.async_copy(idx_hbm.at[pl.ds(base, _ROWS), :], idx_v, sem_i)
    cp_tab.wait()
    cp_idx.wait()

    rows0 = lax.iota(jnp.int32, _L)
    zero16 = jnp.zeros((_L,), jnp.int32)
    rows = [rows0 + (r * _L) for r in range(_GROUPS)]

    def inner(m, accs):
        cols = zero16 + m
        out = []
        for r in range(_GROUPS):
            ids = plsc.load_gather(idx_v, [rows[r], cols])
            vals = plsc.load_gather(tab_v, [ids])
            out.append(accs[r] + jnp.where(ids < _VOCAB, vals,
                                           jnp.float32(0.0)))
        return tuple(out)

    accs = lax.fori_loop(
        0, _M, inner,
        tuple(jnp.zeros((_L,), jnp.float32) for _ in range(_GROUPS)))
    for r in range(_GROUPS):
        out_v[pl.ds(r * _L, _L)] = accs[r]

    pltpu.sync_copy(out_v, out_hbm.at[pl.ds(base, _ROWS)])


@jax.jit
def _sc_call(idx, tab):
    mesh = plsc.VectorSubcoreMesh(core_axis_name="c", subcore_axis_name="s")
    return pl.kernel(
        _sc_body,
        mesh=mesh,
        out_type=jax.ShapeDtypeStruct((_B,), jnp.float32),
        compiler_params=pltpu.CompilerParams(
            needs_layout_passes=False,
            use_tc_tiling_on_sc=False,
        ),
        scratch_types=[
            pltpu.VMEM((_ROWS, _M), jnp.int32),
            pltpu.VMEM((_VOCAB + 1,), jnp.float32),
            pltpu.VMEM((_ROWS,), jnp.float32),
            pltpu.SemaphoreType.DMA,
            pltpu.SemaphoreType.DMA,
        ],
    )(idx, tab)


def kernel(inputs, table):
    return _sc_call(inputs, table.reshape(-1))[:, None]


# speedup vs baseline: 98.8598x; 1.2097x over previous
import jax
import jax.numpy as jnp
from jax import lax
from jax.experimental import pallas as pl
from jax.experimental.pallas import tpu as pltpu
from jax.experimental.pallas import tpu_sc as plsc

_VOCAB = 100000
_B = 4096
_M = 100

_info = plsc.get_sparse_core_info()
_NC, _NS, _L = _info.num_cores, _info.num_subcores, _info.num_lanes
_NW = _NC * _NS
_ROWS = _B // _NW
_GROUPS = _ROWS // _L


def _sc_body(idx_hbm, tab_hbm, out_hbm, idx_v, tab_v, out_v, sem_t, sem_i):
    wid = lax.axis_index("s") * _NC + lax.axis_index("c")
    base = wid * _ROWS

    cp_tab = pltpu.async_copy(tab_hbm, tab_v, sem_t)
    cp_idx = pltpu.async_copy(idx_hbm.at[:, pl.ds(base, _ROWS)], idx_v, sem_i)
    cp_tab.wait()
    cp_idx.wait()

    def inner(m, accs):
        out = []
        for r in range(_GROUPS):
            ids = idx_v[m, pl.ds(r * _L, _L)]
            vals = plsc.load_gather(tab_v, [ids])
            out.append(accs[r] + jnp.where(ids < _VOCAB, vals,
                                           jnp.float32(0.0)))
        return tuple(out)

    accs = lax.fori_loop(
        0, _M, inner,
        tuple(jnp.zeros((_L,), jnp.float32) for _ in range(_GROUPS)))
    for r in range(_GROUPS):
        out_v[pl.ds(r * _L, _L)] = accs[r]

    pltpu.sync_copy(out_v, out_hbm.at[pl.ds(base, _ROWS)])


@jax.jit
def _sc_call(idx_t, tab):
    mesh = plsc.VectorSubcoreMesh(core_axis_name="c", subcore_axis_name="s")
    return pl.kernel(
        _sc_body,
        mesh=mesh,
        out_type=jax.ShapeDtypeStruct((_B,), jnp.float32),
        compiler_params=pltpu.CompilerParams(needs_layout_passes=False),
        scratch_types=[
            pltpu.VMEM((_M, _ROWS), jnp.int32),
            pltpu.VMEM((_VOCAB + 1,), jnp.float32),
            pltpu.VMEM((_ROWS,), jnp.float32),
            pltpu.SemaphoreType.DMA,
            pltpu.SemaphoreType.DMA,
        ],
    )(idx_t, tab)


def kernel(inputs, table):
    return _sc_call(inputs.T, table.reshape(-1))[:, None]
